# Initial kernel scaffold; baseline (speedup 1.0000x reference)
import jax
import jax.numpy as jnp
from jax.experimental import pallas as pl


def kernel(feature_stack_buff, feature_pos, emb_table, W_sb, b_sb, W_pos, b_pos, W_out, b_out):
    raise NotImplementedError("write your pallas kernel here")


# speedup vs baseline: 15.2793x; 15.2793x over previous
import functools

import jax
import jax.numpy as jnp
from jax import lax
from jax.experimental import pallas as pl
from jax.experimental.pallas import tpu as pltpu
from jax.experimental.pallas import tpu_sc as plsc

B = 16384
VOCAB = 1000000
EMB = 32
SB = 128
HID = 128
CLS = 64
HIST = 200

_INFO = plsc.get_sparse_core_info()
_NC = _INFO.num_cores
_NS = _INFO.num_subcores
_NW = _NC * _NS
_RPW = B // _NW
_C = 4
_NCHUNK = _RPW // _C
_SPLIT = 128
_REM = HIST - _SPLIT

_sc_mesh = plsc.VectorSubcoreMesh(core_axis_name="c", subcore_axis_name="s")


@functools.partial(
    pl.kernel,
    out_type=jax.ShapeDtypeStruct((B, EMB), jnp.float32),
    mesh=_sc_mesh,
    compiler_params=pltpu.CompilerParams(use_tc_tiling_on_sc=False),
    scratch_types=[
        pltpu.VMEM((2, _C, HIST), jnp.int32),
        pltpu.VMEM((2, _C * HIST, EMB), jnp.float32),
        pltpu.VMEM((_RPW, EMB), jnp.float32),
        pltpu.SemaphoreType.DMA,
        pltpu.SemaphoreType.DMA,
    ],
)
def _emb_mean(pos_hbm, table_hbm, out_hbm, idx_v, rows_v, stage_v, sem0, sem1):
    wid = lax.axis_index("s") * _NC + lax.axis_index("c")
    base = wid * _RPW
    sems = (sem0, sem1)

    def fire(c, slot):
        pltpu.sync_copy(pos_hbm.at[pl.ds(base + c * _C, _C), :], idx_v.at[slot])
        for r in range(_C):
            pltpu.async_copy(
                table_hbm.at[idx_v.at[slot, r, pl.ds(0, _SPLIT)]],
                rows_v.at[slot, pl.ds(r * HIST, _SPLIT), :],
                sems[slot])
            pltpu.async_copy(
                table_hbm.at[idx_v.at[slot, r, pl.ds(_SPLIT, _REM)]],
                rows_v.at[slot, pl.ds(r * HIST + _SPLIT, _REM), :],
                sems[slot])

    def drain(slot):
        for r in range(_C):
            pltpu.make_async_copy(
                table_hbm.at[idx_v.at[slot, r, pl.ds(0, _SPLIT)]],
                rows_v.at[slot, pl.ds(r * HIST, _SPLIT), :],
                sems[slot]).wait()
            pltpu.make_async_copy(
                table_hbm.at[idx_v.at[slot, r, pl.ds(_SPLIT, _REM)]],
                rows_v.at[slot, pl.ds(r * HIST + _SPLIT, _REM), :],
                sems[slot]).wait()

    inv = jnp.float32(1.0 / HIST)

    def reduce_chunk(c, slot):
        for r in range(_C):
            z = jnp.zeros((16,), jnp.float32)

            def rbody(j, carry, r=r):
                a0, a1, b0, b1 = carry
                p = r * HIST + j * 4
                a0 = a0 + rows_v[slot, p, pl.ds(0, 16)]
                a1 = a1 + rows_v[slot, p, pl.ds(16, 16)]
                b0 = b0 + rows_v[slot, p + 1, pl.ds(0, 16)]
                b1 = b1 + rows_v[slot, p + 1, pl.ds(16, 16)]
                a0 = a0 + rows_v[slot, p + 2, pl.ds(0, 16)]
                a1 = a1 + rows_v[slot, p + 2, pl.ds(16, 16)]
                b0 = b0 + rows_v[slot, p + 3, pl.ds(0, 16)]
                b1 = b1 + rows_v[slot, p + 3, pl.ds(16, 16)]
                return (a0, a1, b0, b1)

            a0, a1, b0, b1 = lax.fori_loop(0, HIST // 4, rbody, (z, z, z, z))
            row = c * _C + r
            stage_v[row, pl.ds(0, 16)] = (a0 + b0) * inv
            stage_v[row, pl.ds(16, 16)] = (a1 + b1) * inv

    fire(0, 0)

    def outer(c2, carry):
        c = c2 * 2
        fire(c + 1, 1)
        drain(0)
        reduce_chunk(c, 0)

        @pl.when(c + 2 < _NCHUNK)
        def _():
            fire(c + 2, 0)

        drain(1)
        reduce_chunk(c + 1, 1)
        return carry

    lax.fori_loop(0, _NCHUNK // 2, outer, 0)
    pltpu.sync_copy(stage_v, out_hbm.at[pl.ds(base, _RPW), :])


_BLK = 1024


def _mlp_body(fsb_ref, emb_ref, wsb_ref, wpos_ref, wout_ref,
              bsb_ref, bpos_ref, bout_ref, out_ref):
    h = jnp.dot(fsb_ref[...], wsb_ref[...], preferred_element_type=jnp.float32)
    h = h + jnp.dot(emb_ref[...], wpos_ref[...], preferred_element_type=jnp.float32)
    h = h + bsb_ref[...] + bpos_ref[...]
    h = jnp.maximum(h, 0.0)
    out_ref[...] = (jnp.dot(h, wout_ref[...], preferred_element_type=jnp.float32)
                    + bout_ref[...])


_mlp = pl.pallas_call(
    _mlp_body,
    grid=(B // _BLK,),
    in_specs=[
        pl.BlockSpec((_BLK, SB), lambda i: (i, 0)),
        pl.BlockSpec((_BLK, EMB), lambda i: (i, 0)),
        pl.BlockSpec((SB, HID), lambda i: (0, 0)),
        pl.BlockSpec((EMB, HID), lambda i: (0, 0)),
        pl.BlockSpec((HID, CLS), lambda i: (0, 0)),
        pl.BlockSpec((1, HID), lambda i: (0, 0)),
        pl.BlockSpec((1, HID), lambda i: (0, 0)),
        pl.BlockSpec((1, CLS), lambda i: (0, 0)),
    ],
    out_specs=pl.BlockSpec((_BLK, CLS), lambda i: (i, 0)),
    out_shape=jax.ShapeDtypeStruct((B, CLS), jnp.float32),
)


def kernel(feature_stack_buff, feature_pos, emb_table,
           W_sb, b_sb, W_pos, b_pos, W_out, b_out):
    pos = feature_pos.astype(jnp.int32)
    emb_mean = _emb_mean(pos, emb_table)
    return _mlp(feature_stack_buff, emb_mean,
                W_sb, W_pos, W_out,
                b_sb.reshape(1, HID), b_pos.reshape(1, HID),
                b_out.reshape(1, CLS))


# speedup vs baseline: 15.2884x; 1.0006x over previous
import functools

import jax
import jax.numpy as jnp
from jax import lax
from jax.experimental import pallas as pl
from jax.experimental.pallas import tpu as pltpu
from jax.experimental.pallas import tpu_sc as plsc

B = 16384
VOCAB = 1000000
EMB = 32
SB = 128
HID = 128
CLS = 64
HIST = 200

_INFO = plsc.get_sparse_core_info()
_NC = _INFO.num_cores
_NS = _INFO.num_subcores
_NW = _NC * _NS
_RPW = B // _NW
_C = 4
_NCHUNK = _RPW // _C
_SPLIT = 128
_REM = HIST - _SPLIT

_sc_mesh = plsc.VectorSubcoreMesh(core_axis_name="c", subcore_axis_name="s")


@functools.partial(
    pl.kernel,
    out_type=jax.ShapeDtypeStruct((B, EMB), jnp.float32),
    name="emb_mean_sc",
    mesh=_sc_mesh,
    compiler_params=pltpu.CompilerParams(use_tc_tiling_on_sc=False),
    scratch_types=[
        pltpu.VMEM((2, _C * HIST), jnp.int32),
        pltpu.VMEM((2, _C * HIST, EMB), jnp.float32),
        pltpu.VMEM((_RPW, EMB), jnp.float32),
        pltpu.SemaphoreType.DMA,
        pltpu.SemaphoreType.DMA,
    ],
)
def _emb_mean(pos_hbm, table_hbm, out_hbm, idx_v, rows_v, stage_v, sem0, sem1):
    wid = lax.axis_index("s") * _NC + lax.axis_index("c")
    base = wid * _RPW
    sems = (sem0, sem1)

    def fire(c, slot):
        pltpu.sync_copy(pos_hbm.at[pl.ds((base + c * _C) * HIST, _C * HIST)],
                        idx_v.at[slot])
        for r in range(_C):
            pltpu.async_copy(
                table_hbm.at[idx_v.at[slot, pl.ds(r * HIST, _SPLIT)]],
                rows_v.at[slot, pl.ds(r * HIST, _SPLIT), :],
                sems[slot])
            pltpu.async_copy(
                table_hbm.at[idx_v.at[slot, pl.ds(r * HIST + _SPLIT, _REM)]],
                rows_v.at[slot, pl.ds(r * HIST + _SPLIT, _REM), :],
                sems[slot])

    def drain(slot):
        for r in range(_C):
            pltpu.make_async_copy(
                table_hbm.at[idx_v.at[slot, pl.ds(r * HIST, _SPLIT)]],
                rows_v.at[slot, pl.ds(r * HIST, _SPLIT), :],
                sems[slot]).wait()
            pltpu.make_async_copy(
                table_hbm.at[idx_v.at[slot, pl.ds(r * HIST + _SPLIT, _REM)]],
                rows_v.at[slot, pl.ds(r * HIST + _SPLIT, _REM), :],
                sems[slot]).wait()

    inv = jnp.float32(1.0 / HIST)

    def reduce_chunk(c, slot):
        for r in range(_C):
            z = jnp.zeros((16,), jnp.float32)

            def rbody(j, carry, r=r):
                a0, a1, b0, b1 = carry
                p = r * HIST + j * 4
                a0 = a0 + rows_v[slot, p, pl.ds(0, 16)]
                a1 = a1 + rows_v[slot, p, pl.ds(16, 16)]
                b0 = b0 + rows_v[slot, p + 1, pl.ds(0, 16)]
                b1 = b1 + rows_v[slot, p + 1, pl.ds(16, 16)]
                a0 = a0 + rows_v[slot, p + 2, pl.ds(0, 16)]
                a1 = a1 + rows_v[slot, p + 2, pl.ds(16, 16)]
                b0 = b0 + rows_v[slot, p + 3, pl.ds(0, 16)]
                b1 = b1 + rows_v[slot, p + 3, pl.ds(16, 16)]
                return (a0, a1, b0, b1)

            a0, a1, b0, b1 = lax.fori_loop(0, HIST // 4, rbody, (z, z, z, z))
            row = c * _C + r
            stage_v[row, pl.ds(0, 16)] = (a0 + b0) * inv
            stage_v[row, pl.ds(16, 16)] = (a1 + b1) * inv

    fire(0, 0)

    def outer(c2, carry):
        c = c2 * 2
        fire(c + 1, 1)
        drain(0)
        reduce_chunk(c, 0)

        @pl.when(c + 2 < _NCHUNK)
        def _():
            fire(c + 2, 0)

        drain(1)
        reduce_chunk(c + 1, 1)
        return carry

    lax.fori_loop(0, _NCHUNK // 2, outer, 0)
    pltpu.sync_copy(stage_v, out_hbm.at[pl.ds(base, _RPW), :])


_BLK = 1024


def _mlp_body(fsb_ref, emb_ref, wsb_ref, wpos_ref, wout_ref,
              bsb_ref, bpos_ref, bout_ref, out_ref):
    h = jnp.dot(fsb_ref[...], wsb_ref[...], preferred_element_type=jnp.float32)
    h = h + jnp.dot(emb_ref[...], wpos_ref[...], preferred_element_type=jnp.float32)
    h = h + bsb_ref[...] + bpos_ref[...]
    h = jnp.maximum(h, 0.0)
    out_ref[...] = (jnp.dot(h, wout_ref[...], preferred_element_type=jnp.float32)
                    + bout_ref[...])


_mlp = pl.pallas_call(
    _mlp_body,
    grid=(B // _BLK,),
    in_specs=[
        pl.BlockSpec((_BLK, SB), lambda i: (i, 0)),
        pl.BlockSpec((_BLK, EMB), lambda i: (i, 0)),
        pl.BlockSpec((SB, HID), lambda i: (0, 0)),
        pl.BlockSpec((EMB, HID), lambda i: (0, 0)),
        pl.BlockSpec((HID, CLS), lambda i: (0, 0)),
        pl.BlockSpec((1, HID), lambda i: (0, 0)),
        pl.BlockSpec((1, HID), lambda i: (0, 0)),
        pl.BlockSpec((1, CLS), lambda i: (0, 0)),
    ],
    out_specs=pl.BlockSpec((_BLK, CLS), lambda i: (i, 0)),
    out_shape=jax.ShapeDtypeStruct((B, CLS), jnp.float32),
)


def kernel(feature_stack_buff, feature_pos, emb_table,
           W_sb, b_sb, W_pos, b_pos, W_out, b_out):
    pos = feature_pos.reshape(-1).astype(jnp.int32)
    emb_mean = _emb_mean(pos, emb_table)
    return _mlp(feature_stack_buff, emb_mean,
                W_sb, W_pos, W_out,
                b_sb.reshape(1, HID), b_pos.reshape(1, HID),
                b_out.reshape(1, CLS))


# speedup vs baseline: 15.8141x; 1.0344x over previous
import functools

import jax
import jax.numpy as jnp
from jax import lax
from jax.experimental import pallas as pl
from jax.experimental.pallas import tpu as pltpu
from jax.experimental.pallas import tpu_sc as plsc

B = 16384
VOCAB = 1000000
EMB = 32
SB = 128
HID = 128
CLS = 64
HIST = 200

_INFO = plsc.get_sparse_core_info()
_NC = _INFO.num_cores
_NS = _INFO.num_subcores
_NW = _NC * _NS
_RPW = B // _NW
_C = 4
_NCHUNK = _RPW // _C
_SPLIT = 128
_REM = HIST - _SPLIT

_CPS = 32
_NSUP = _NCHUNK // _CPS
_IDXSUP = _CPS * _C * HIST

_sc_mesh = plsc.VectorSubcoreMesh(core_axis_name="c", subcore_axis_name="s")


@functools.partial(
    pl.kernel,
    out_type=jax.ShapeDtypeStruct((B, EMB), jnp.float32),
    name="emb_mean_sc",
    mesh=_sc_mesh,
    compiler_params=pltpu.CompilerParams(use_tc_tiling_on_sc=False),
    scratch_types=[
        pltpu.VMEM((2, _IDXSUP), jnp.int32),
        pltpu.VMEM((2, _C * HIST, EMB), jnp.float32),
        pltpu.VMEM((_RPW, EMB), jnp.float32),
        pltpu.SemaphoreType.DMA,
        pltpu.SemaphoreType.DMA,
        pltpu.SemaphoreType.DMA,
    ],
)
def _emb_mean(pos_hbm, table_hbm, out_hbm, idx_v, rows_v, stage_v,
              sem0, sem1, semi):
    wid = lax.axis_index("s") * _NC + lax.axis_index("c")
    base = wid * _RPW
    pos_base = base * HIST
    sems = (sem0, sem1)

    def streams(lc, islot, rslot):
        out = []
        for r in range(_C):
            out.append(pltpu.make_async_copy(
                table_hbm.at[idx_v.at[islot, pl.ds(lc * _C * HIST + r * HIST, _SPLIT)]],
                rows_v.at[rslot, pl.ds(r * HIST, _SPLIT), :],
                sems[rslot]))
            out.append(pltpu.make_async_copy(
                table_hbm.at[idx_v.at[islot, pl.ds(lc * _C * HIST + r * HIST + _SPLIT, _REM)]],
                rows_v.at[rslot, pl.ds(r * HIST + _SPLIT, _REM), :],
                sems[rslot]))
        return out

    def fire(lc, islot, rslot):
        for cp in streams(lc, islot, rslot):
            cp.start()

    def drain(lc, islot, rslot):
        for cp in streams(lc, islot, rslot):
            cp.wait()

    inv = jnp.float32(1.0 / HIST)

    def reduce_chunk(gc, rslot):
        for r in range(_C):
            z = jnp.zeros((16,), jnp.float32)

            def rbody(j, carry, r=r):
                a0, a1, b0, b1 = carry
                p = r * HIST + j * 4
                a0 = a0 + rows_v[rslot, p, pl.ds(0, 16)]
                a1 = a1 + rows_v[rslot, p, pl.ds(16, 16)]
                b0 = b0 + rows_v[rslot, p + 1, pl.ds(0, 16)]
                b1 = b1 + rows_v[rslot, p + 1, pl.ds(16, 16)]
                a0 = a0 + rows_v[rslot, p + 2, pl.ds(0, 16)]
                a1 = a1 + rows_v[rslot, p + 2, pl.ds(16, 16)]
                b0 = b0 + rows_v[rslot, p + 3, pl.ds(0, 16)]
                b1 = b1 + rows_v[rslot, p + 3, pl.ds(16, 16)]
                return (a0, a1, b0, b1)

            a0, a1, b0, b1 = lax.fori_loop(0, HIST // 4, rbody, (z, z, z, z))
            row = gc * _C + r
            stage_v[row, pl.ds(0, 16)] = (a0 + b0) * inv
            stage_v[row, pl.ds(16, 16)] = (a1 + b1) * inv

    def idx_copy(s, islot):
        return pltpu.make_async_copy(
            pos_hbm.at[pl.ds(pos_base + s * _IDXSUP, _IDXSUP)],
            idx_v.at[islot], semi)

    idx_copy(0, 0).start()
    idx_copy(0, 0).wait()

    for s in range(_NSUP):
        islot = s % 2
        if s > 0:
            idx_copy(s, islot).wait()
        if s + 1 < _NSUP:
            idx_copy(s + 1, 1 - islot).start()
        fire(0, islot, 0)

        def pair(t, carry, s=s, islot=islot):
            lc = 2 * t
            gc = s * _CPS + lc
            fire(lc + 1, islot, 1)
            drain(lc, islot, 0)
            reduce_chunk(gc, 0)
            fire(lc + 2, islot, 0)
            drain(lc + 1, islot, 1)
            reduce_chunk(gc + 1, 1)
            return carry

        lax.fori_loop(0, _CPS // 2 - 1, pair, 0)

        lc = _CPS - 2
        gc = s * _CPS + lc
        fire(lc + 1, islot, 1)
        drain(lc, islot, 0)
        reduce_chunk(gc, 0)
        drain(lc + 1, islot, 1)
        reduce_chunk(gc + 1, 1)

    pltpu.sync_copy(stage_v, out_hbm.at[pl.ds(base, _RPW), :])


_BLK = 1024


def _mlp_body(fsb_ref, emb_ref, wsb_ref, wpos_ref, wout_ref,
              bsb_ref, bpos_ref, bout_ref, out_ref):
    h = jnp.dot(fsb_ref[...], wsb_ref[...], preferred_element_type=jnp.float32)
    h = h + jnp.dot(emb_ref[...], wpos_ref[...], preferred_element_type=jnp.float32)
    h = h + bsb_ref[...] + bpos_ref[...]
    h = jnp.maximum(h, 0.0)
    out_ref[...] = (jnp.dot(h, wout_ref[...], preferred_element_type=jnp.float32)
                    + bout_ref[...])


_mlp = pl.pallas_call(
    _mlp_body,
    grid=(B // _BLK,),
    in_specs=[
        pl.BlockSpec((_BLK, SB), lambda i: (i, 0)),
        pl.BlockSpec((_BLK, EMB), lambda i: (i, 0)),
        pl.BlockSpec((SB, HID), lambda i: (0, 0)),
        pl.BlockSpec((EMB, HID), lambda i: (0, 0)),
        pl.BlockSpec((HID, CLS), lambda i: (0, 0)),
        pl.BlockSpec((1, HID), lambda i: (0, 0)),
        pl.BlockSpec((1, HID), lambda i: (0, 0)),
        pl.BlockSpec((1, CLS), lambda i: (0, 0)),
    ],
    out_specs=pl.BlockSpec((_BLK, CLS), lambda i: (i, 0)),
    out_shape=jax.ShapeDtypeStruct((B, CLS), jnp.float32),
)


def kernel(feature_stack_buff, feature_pos, emb_table,
           W_sb, b_sb, W_pos, b_pos, W_out, b_out):
    pos = feature_pos.reshape(-1).astype(jnp.int32)
    emb_mean = _emb_mean(pos, emb_table)
    return _mlp(feature_stack_buff, emb_mean,
                W_sb, W_pos, W_out,
                b_sb.reshape(1, HID), b_pos.reshape(1, HID),
                b_out.reshape(1, CLS))
